# Initial kernel scaffold; baseline (speedup 1.0000x reference)
import jax
import jax.numpy as jnp
from jax.experimental import pallas as pl


def kernel(x, edge_index, edge_weight, W1, b1, W2, b2, Wd1, bd1, Wd2, bd2):
    raise NotImplementedError("write your pallas kernel here")


# speedup vs baseline: 4.9600x; 4.9600x over previous
import functools

import jax
import jax.numpy as jnp
from jax import lax
from jax.experimental import pallas as pl
from jax.experimental.pallas import tpu as pltpu
from jax.experimental.pallas import tpu_sc as plsc

N = 10000
NP = 10240
E = 160000
NSUB = 16
NCORES = 2
EPT_SC = E // NSUB
EPT = E // (NCORES * NSUB)
ROWS_PT = NP // NSUB
SROWS_PT = NP // NSUB
DROW = 128
CHW = 128
BATCH = 128
EPTA = E // NSUB
NBATCHA = 78
NGRP = BATCH // 16
ZROW = 32
HALF = 5120
SLABR = 5248
DUMPR = HALF
RB = 1000
GRID = N // RB

_mesh = lambda: plsc.VectorSubcoreMesh(core_axis_name="c", subcore_axis_name="s")
_params = lambda: pltpu.CompilerParams(needs_layout_passes=False)


def _rsqrt16(x):
    i = plsc.bitcast(x, jnp.int32)
    magic = jnp.full((16,), 0x5F3759DF, dtype=jnp.int32)
    y = plsc.bitcast(magic - lax.shift_right_logical(i, 1), jnp.float32)
    half = jnp.full((16,), 0.5, jnp.float32)
    th = jnp.full((16,), 1.5, jnp.float32)
    for _ in range(3):
        y = y * (th - half * x * y * y)
    return jnp.where(x > 0.0, y, jnp.zeros((16,), jnp.float32))


def _sc_pre_body(deg_hbm, src_hbm, dst_hbm, ew_hbm, norm_hbm, dis2_hbm,
                 degbuf_v, dis_v, dtmp_v, sidx_v, didx_v, ew2_v, nrm_v,
                 d2_v, dis_sh):
    c = lax.axis_index("c")
    s = lax.axis_index("s")

    iota16 = lax.iota(jnp.int32, 16)
    zero16i = jnp.zeros((16,), jnp.int32)
    ones16 = jnp.full((16,), 1.0, jnp.float32)

    cb = pl.multiple_of(s * ROWS_PT, 8)
    for p in range(ROWS_PT // 64):
        pltpu.sync_copy(deg_hbm.at[pl.ds(cb + p * 64, 64)], degbuf_v)

        def disb(j, _, p=p):
            rows = iota16 + lax.broadcast(j * 16, (16,))
            deg16 = plsc.load_gather(degbuf_v, [rows, zero16i])
            dtmp_v[pl.ds(p * 64 + j * 16, 16)] = _rsqrt16(deg16 + ones16)
            return 0
        lax.fori_loop(0, 4, disb, 0)
    pltpu.sync_copy(dtmp_v, dis_sh.at[pl.ds(cb, ROWS_PT)])
    plsc.subcore_barrier()
    pltpu.sync_copy(dis_sh, dis_v)

    geb = pl.multiple_of((c * NSUB + s) * EPT, 8)
    pltpu.sync_copy(src_hbm.at[pl.ds(geb, EPT)], sidx_v)
    pltpu.sync_copy(dst_hbm.at[pl.ds(geb, EPT)], didx_v)
    pltpu.sync_copy(ew_hbm.at[pl.ds(geb, EPT)], ew2_v)

    def nb(i, _):
        base = jnp.minimum(i * 16, EPT - 16)
        s16 = sidx_v[pl.ds(base, 16)]
        d16 = didx_v[pl.ds(base, 16)]
        w16 = ew2_v[pl.ds(base, 16)]
        nrm_v[pl.ds(base, 16)] = (plsc.load_gather(dis_v, [s16]) * w16 *
                                  plsc.load_gather(dis_v, [d16]))
        return 0
    lax.fori_loop(0, (EPT + 15) // 16, nb, 0)
    pltpu.sync_copy(nrm_v, norm_hbm.at[pl.ds(geb, EPT)])

    @pl.when(c == 0)
    def _():
        def d2b(j, _):
            d16 = dis_v[pl.ds(cb + j * 16, 16)]
            d2_v[pl.ds(j * 16, 16)] = d16 * d16
            return 0
        lax.fori_loop(0, ROWS_PT // 16, d2b, 0)
        pltpu.sync_copy(d2_v, dis2_hbm.at[pl.ds(cb, ROWS_PT)])


_sc_pre = pl.kernel(
    _sc_pre_body,
    out_type=(jax.ShapeDtypeStruct((E,), jnp.float32),
              jax.ShapeDtypeStruct((NP,), jnp.float32)),
    mesh=_mesh(),
    compiler_params=_params(),
    scratch_types=[
        pltpu.VMEM((64, CHW), jnp.float32),
        pltpu.VMEM((NP,), jnp.float32),
        pltpu.VMEM((ROWS_PT,), jnp.float32),
        pltpu.VMEM((EPT,), jnp.int32),
        pltpu.VMEM((EPT,), jnp.int32),
        pltpu.VMEM((EPT,), jnp.float32),
        pltpu.VMEM((EPT,), jnp.float32),
        pltpu.VMEM((ROWS_PT,), jnp.float32),
        pltpu.VMEM_SHARED((NP,), jnp.float32),
    ],
)


def _sc_agg_body(tab_hbm, src_hbm, dst_hbm, nrm_hbm, out_hbm, *scr):
    srcv, nrmv, msg0, msg1, zb, dstraw, shbuf, dstg, slab, sem0, sem1 = scr
    c = lax.axis_index("c")
    s = lax.axis_index("s")
    geb = pl.multiple_of(s * EPTA, 8)
    pltpu.sync_copy(src_hbm.at[pl.ds(geb, EPTA)], srcv)
    pltpu.sync_copy(nrm_hbm.at[pl.ds(geb, EPTA)], nrmv)

    zeros16 = jnp.zeros((16,), jnp.float32)

    def zbody(i, _):
        for k in range(CHW // 16):
            zb[i, pl.ds(k * 16, 16)] = zeros16
        return 0
    lax.fori_loop(0, ZROW, zbody, 0)

    for j in range(11):
        blk = s + 16 * j

        @pl.when(blk < SLABR // ZROW)
        def _():
            pltpu.sync_copy(zb, slab.at[pl.ds(pl.multiple_of(blk * ZROW, 8),
                                              ZROW)])
    plsc.subcore_barrier()

    cbase16 = lax.broadcast(c * HALF, (16,))
    dump16 = lax.broadcast(DUMPR, (16,)) + lax.broadcast(s, (16,))
    iota16 = lax.iota(jnp.int32, 16)
    prev_idx = jnp.maximum(iota16 - 1, 0)
    lo16 = jnp.zeros((16,), jnp.int32)
    hi16 = jnp.full((16,), HALF, jnp.int32)
    one16 = jnp.full((16,), 1, jnp.int32)
    zero16i = jnp.zeros((16,), jnp.int32)
    lane0 = iota16 == zero16i

    def gather(a, buf, sem):
        return pltpu.async_copy(
            tab_hbm.at[srcv.at[pl.ds(pl.multiple_of(a * BATCH, 8), BATCH)]],
            buf, sem)

    def scale(boff, buf, ng):
        def sbody(g, _):
            nv = nrmv[pl.ds(boff + g * 16, 16)]
            for lane in range(16):
                bs = lax.broadcast(nv[lane], (16,))
                row = g * 16 + lane
                for k in range(CHW // 16):
                    buf[row, pl.ds(k * 16, 16)] = (
                        buf[row, pl.ds(k * 16, 16)] * bs)
            return 0
        lax.fori_loop(0, ng, sbody, 0)

    def scatter_groups(boff, buf, ng):
        pltpu.sync_copy(dst_hbm.at[pl.ds(pl.multiple_of(geb + boff, 8),
                                         16 * ng)],
                        dstraw.at[pl.ds(0, 16 * ng)])

        def gbody(g, _):
            gb = pl.multiple_of(g * 16, 8)
            d16 = dstraw[pl.ds(gb, 16)] - cbase16
            okm = (d16 >= lo16) & (d16 < hi16)
            d16 = jnp.where(okm, d16, dump16)
            dstg[...] = d16
            pltpu.sync_copy(buf.at[pl.ds(gb, 16)], slab.at[dstg], add=True)
            return 0
        lax.fori_loop(0, ng, gbody, 0)

    cp = gather(0, msg0, sem0)

    def pair(t, _):
        a0 = 2 * t
        gather(a0 + 1, msg1, sem1)
        pltpu.make_async_copy(tab_hbm.at[srcv.at[pl.ds(0, BATCH)]],
                              msg0, sem0).wait()
        scale(a0 * BATCH, msg0, NGRP)
        scatter_groups(a0 * BATCH, msg0, NGRP)
        gather(jnp.minimum(a0 + 2, NBATCHA - 2), msg0, sem0)
        pltpu.make_async_copy(tab_hbm.at[srcv.at[pl.ds(0, BATCH)]],
                              msg1, sem1).wait()
        scale((a0 + 1) * BATCH, msg1, NGRP)
        scatter_groups((a0 + 1) * BATCH, msg1, NGRP)
        return 0
    lax.fori_loop(0, NBATCHA // 2, pair, 0)
    pltpu.make_async_copy(tab_hbm.at[srcv.at[pl.ds(0, BATCH)]],
                          msg0, sem0).wait()

    eoff = NBATCHA * BATCH
    pltpu.sync_copy(tab_hbm.at[srcv.at[pl.ds(eoff, 16)]],
                    msg0.at[pl.ds(0, 16)])
    scale(eoff, msg0, 1)
    scatter_groups(eoff, msg0, 1)

    plsc.subcore_barrier()
    for (off, nrow) in ((0, 128), (128, 128), (256, 64)):
        ro = pl.multiple_of(s * (HALF // NSUB) + off, 8)
        pltpu.sync_copy(slab.at[pl.ds(ro, nrow)],
                        out_hbm.at[pl.ds(pl.multiple_of(c * HALF + ro, 8),
                                         nrow)])


_sc_agg1 = pl.kernel(
    _sc_agg_body,
    out_type=jax.ShapeDtypeStruct((NP, CHW), jnp.float32),
    mesh=_mesh(),
    compiler_params=_params(),
    scratch_types=[
        pltpu.VMEM((EPTA,), jnp.int32),
        pltpu.VMEM((EPTA,), jnp.float32),
        pltpu.VMEM((BATCH, CHW), jnp.float32),
        pltpu.VMEM((BATCH, CHW), jnp.float32),
        pltpu.VMEM((ZROW, CHW), jnp.float32),
        pltpu.VMEM((BATCH,), jnp.int32),
        pltpu.VMEM((16,), jnp.int32),
        pltpu.VMEM((16,), jnp.int32),
        pltpu.VMEM_SHARED((SLABR, CHW), jnp.float32),
        pltpu.SemaphoreType.DMA,
        pltpu.SemaphoreType.DMA,
    ],
)


def _tc1_body(x_ref, w_ref, o_ref):
    acc = jnp.dot(x_ref[...], w_ref[...], preferred_element_type=jnp.float32)
    for ci in range(4):
        o_ref[ci] = acc[:, ci * CHW:(ci + 1) * CHW]


_tc1 = pl.pallas_call(
    _tc1_body,
    grid=(GRID,),
    in_specs=[pl.BlockSpec((RB, 256), lambda i: (i, 0)),
              pl.BlockSpec((256, 512), lambda i: (0, 0))],
    out_specs=pl.BlockSpec((4, RB, CHW), lambda i: (0, i, 0)),
    out_shape=jax.ShapeDtypeStruct((4, N, CHW), jnp.float32),
)


def _tc2_body(s0, s1, s2, s3, h_ref, dis2_ref, b1_ref, w2_ref, o_ref):
    dis2 = dis2_ref[...]
    parts = []
    for ci, sr in enumerate((s0, s1, s2, s3)):
        parts.append(sr[...] + dis2 * h_ref[ci]
                     + b1_ref[:, ci * CHW:(ci + 1) * CHW])
    h1 = jnp.maximum(jnp.concatenate(parts, axis=1), 0.0)
    acc = jnp.dot(h1, w2_ref[...], preferred_element_type=jnp.float32)
    for ci in range(2):
        o_ref[ci] = acc[:, ci * CHW:(ci + 1) * CHW]


_tc2 = pl.pallas_call(
    _tc2_body,
    grid=(GRID,),
    in_specs=[pl.BlockSpec((RB, CHW), lambda i: (i, 0)),
              pl.BlockSpec((RB, CHW), lambda i: (i, 0)),
              pl.BlockSpec((RB, CHW), lambda i: (i, 0)),
              pl.BlockSpec((RB, CHW), lambda i: (i, 0)),
              pl.BlockSpec((4, RB, CHW), lambda i: (0, i, 0)),
              pl.BlockSpec((RB, 1), lambda i: (i, 0)),
              pl.BlockSpec((1, 512), lambda i: (0, 0)),
              pl.BlockSpec((512, 256), lambda i: (0, 0))],
    out_specs=pl.BlockSpec((2, RB, CHW), lambda i: (0, i, 0)),
    out_shape=jax.ShapeDtypeStruct((2, N, CHW), jnp.float32),
)


def _tc3_body(s0, s1, z_ref, dis2_ref, b2_ref, wd1_ref, bd1_ref,
              wd2_ref, bd2_ref, o_ref):
    dis2 = dis2_ref[...]
    parts = []
    for ci, sr in enumerate((s0, s1)):
        parts.append(sr[...] + dis2 * z_ref[ci]
                     + b2_ref[:, ci * CHW:(ci + 1) * CHW])
    z = jnp.maximum(jnp.concatenate(parts, axis=1), 0.0)
    d = jnp.maximum(
        jnp.dot(z, wd1_ref[...], preferred_element_type=jnp.float32)
        + bd1_ref[...], 0.0)
    t = jnp.dot(d, wd2_ref[...], preferred_element_type=jnp.float32) + bd2_ref[...]
    o_ref[...] = 1.0 / (1.0 + jnp.exp(-t))


_tc3 = pl.pallas_call(
    _tc3_body,
    grid=(GRID,),
    in_specs=[pl.BlockSpec((RB, CHW), lambda i: (i, 0)),
              pl.BlockSpec((RB, CHW), lambda i: (i, 0)),
              pl.BlockSpec((2, RB, CHW), lambda i: (0, i, 0)),
              pl.BlockSpec((RB, 1), lambda i: (i, 0)),
              pl.BlockSpec((1, 256), lambda i: (0, 0)),
              pl.BlockSpec((256, 512), lambda i: (0, 0)),
              pl.BlockSpec((1, 512), lambda i: (0, 0)),
              pl.BlockSpec((512, 256), lambda i: (0, 0)),
              pl.BlockSpec((1, 256), lambda i: (0, 0))],
    out_specs=pl.BlockSpec((RB, 256), lambda i: (i, 0)),
    out_shape=jax.ShapeDtypeStruct((N, 256), jnp.float32),
)


_ONES_TAB = jnp.ones((N, CHW), jnp.float32)


def kernel(x, edge_index, edge_weight, W1, b1, W2, b2, Wd1, bd1, Wd2, bd2):
    src = edge_index[0].astype(jnp.int32)
    dst = edge_index[1].astype(jnp.int32)
    ew = edge_weight.astype(jnp.float32)

    deg_tab = _sc_agg1(_ONES_TAB, src, dst, ew)
    norm, dis2p = _sc_pre(deg_tab, src, dst, ew)
    dis2 = dis2p[:N].reshape(N, 1)

    h_tab = _tc1(x, W1)
    sl1 = [_sc_agg1(h_tab[ci], src, dst, norm) for ci in range(4)]
    z_tab = _tc2(*sl1, h_tab, dis2, b1.reshape(1, 512), W2)
    sl2 = [_sc_agg1(z_tab[ci], src, dst, norm) for ci in range(2)]
    return _tc3(*sl2, z_tab, dis2, b2.reshape(1, 256),
                Wd1, bd1.reshape(1, 512), Wd2, bd2.reshape(1, 256))
